# baseline (speedup 1.0000x reference)
import jax
import jax.numpy as jnp
from jax import lax
from jax.experimental import pallas as pl
from jax.experimental.pallas import tpu as pltpu
from jax.experimental.pallas import tpu_sc as plsc

VOCAB = 1000000
DIM = 64
BATCH = 16384

NUM_CORES = 2
NUM_SUBCORES = 16
NUM_WORKERS = NUM_CORES * NUM_SUBCORES
BPW = BATCH // NUM_WORKERS
IDX_CHUNK = 128
NCHUNK = BPW // IDX_CHUNK
LANES = 16


def _sc_kernel_body(idx0_hbm, idx1_hbm, labels_hbm, t0_hbm, t1_hbm,
                    out_hbm, idx0_v, idx1_v, lab_v, rows0_v, rows1_v,
                    part_v, sem):
    wid = lax.axis_index("s") * NUM_CORES + lax.axis_index("c")
    base = wid * BPW
    chunk_base = wid * NCHUNK

    pltpu.sync_copy(idx0_hbm.at[pl.ds(chunk_base, NCHUNK)], idx0_v)
    pltpu.sync_copy(idx1_hbm.at[pl.ds(chunk_base, NCHUNK)], idx1_v)
    pltpu.sync_copy(labels_hbm.at[pl.ds(base, BPW)], lab_v)

    copies = []
    for j in range(NCHUNK):
        copies.append(pltpu.async_copy(
            t0_hbm.at[idx0_v.at[j]],
            rows0_v.at[pl.ds(j * IDX_CHUNK, IDX_CHUNK)], sem))
        copies.append(pltpu.async_copy(
            t1_hbm.at[idx1_v.at[j]],
            rows1_v.at[pl.ds(j * IDX_CHUNK, IDX_CHUNK)], sem))
    for c in copies:
        c.wait()

    lane = lax.broadcasted_iota(jnp.int32, (LANES,), 0)

    def group_step(g, loss_acc):
        rows = lane + g * LANES

        def d_step(d, acc):
            col = jnp.full((LANES,), d, jnp.int32)
            v0 = plsc.load_gather(rows0_v, [rows, col])
            v1 = plsc.load_gather(rows1_v, [rows, col])
            return acc + v0 * v1

        pred = lax.fori_loop(0, DIM, d_step,
                             jnp.zeros((LANES,), jnp.float32))
        sig = 1.0 / (1.0 + jnp.exp(-pred))
        diff = sig - lab_v[pl.ds(g * LANES, LANES)]
        return loss_acc + diff * diff

    loss_acc = lax.fori_loop(0, BPW // LANES, group_step,
                             jnp.zeros((LANES,), jnp.float32))

    part_v[...] = loss_acc
    pltpu.sync_copy(part_v, out_hbm.at[wid])


@jax.jit
def _run(idx0, idx1, labels, t0, t1):
    mesh = plsc.VectorSubcoreMesh(core_axis_name="c", subcore_axis_name="s")
    partials = pl.kernel(
        _sc_kernel_body,
        out_type=jax.ShapeDtypeStruct((NUM_WORKERS, LANES), jnp.float32),
        mesh=mesh,
        compiler_params=pltpu.CompilerParams(
            needs_layout_passes=False, use_tc_tiling_on_sc=False),
        scratch_types=[
            pltpu.VMEM((NCHUNK, IDX_CHUNK), jnp.int32),
            pltpu.VMEM((NCHUNK, IDX_CHUNK), jnp.int32),
            pltpu.VMEM((BPW,), jnp.float32),
            pltpu.VMEM((BPW, DIM), jnp.float32),
            pltpu.VMEM((BPW, DIM), jnp.float32),
            pltpu.VMEM((LANES,), jnp.float32),
            pltpu.SemaphoreType.DMA,
        ],
    )(idx0, idx1, labels, t0, t1)
    return jnp.sum(partials) * (1.0 / BATCH)


def kernel(indices_f0, indices_f1, labels, emb_table_0, emb_table_1):
    idx0 = indices_f0.astype(jnp.int32).reshape(NUM_WORKERS * NCHUNK,
                                                IDX_CHUNK)
    idx1 = indices_f1.astype(jnp.int32).reshape(NUM_WORKERS * NCHUNK,
                                                IDX_CHUNK)
    return _run(idx0, idx1, labels, emb_table_0, emb_table_1)
